# baseline (speedup 1.0000x reference)
import functools

import jax
import jax.numpy as jnp
from jax import lax
from jax.experimental import pallas as pl
from jax.experimental.pallas import tpu as pltpu
from jax.experimental.pallas import tpu_sc as plsc

VOCAB = 1000000
EMB = 64
BATCH = 16384
CTX = 5
LANES = 16

_info = plsc.get_sparse_core_info()
_NC, _NS = _info.num_cores, _info.num_subcores
NW = _NC * _NS
BPW = BATCH // NW
CB = 128
NCHUNK = BPW // CB
VT = VOCAB // 128
TAIL = VOCAB - VT * 128
PR = VOCAB // 2

_params = pltpu.CompilerParams(
    needs_layout_passes=False, use_tc_tiling_on_sc=True)
_mesh = plsc.VectorSubcoreMesh(core_axis_name="c", subcore_axis_name="s")


def _tr_body(ttab3, ttail, ctab3, ctail, scr_t, scr_c,
             in_v, out_v, tail_v, sem_in, sem_out):
    wid = lax.axis_index("s") * _NC + lax.axis_index("c")
    lane = lax.iota(jnp.int32, LANES)
    te = [(k * LANES + lane) >> 3 for k in range(EMB // LANES)]
    es = [(k * LANES + lane) & 7 for k in range(EMB // LANES)]
    nk = jnp.where(wid < VT % NW, VT // NW + 1, VT // NW)

    for tab3, tail, scr in ((ttab3, ttail, scr_t), (ctab3, ctail, scr_c)):
        def fire(k, slot, tab3=tab3):
            tv = wid + NW * k
            off = pl.multiple_of(tv * 128, 128)
            pltpu.async_copy(tab3.at[:, :, pl.ds(off, 128)],
                             in_v.at[slot], sem_in.at[slot])

        fire(0, 0)

        def step(g, carry, tab3=tab3, scr=scr):
            s = lax.rem(g, 2)
            sn = 1 - s

            @pl.when(g + 1 < nk)
            def _():
                fire(g + 1, sn)

            pltpu.make_async_copy(
                tab3.at[:, :, pl.ds(0, 128)], in_v.at[s],
                sem_in.at[s]).wait()

            @pl.when(g >= 2)
            def _():
                pltpu.make_async_copy(
                    out_v.at[s], scr.at[pl.ds(0, EMB)],
                    sem_out.at[s]).wait()

            def row(i, c, in_v=in_v):
                col = jnp.full((LANES,), 2 * i + 1, jnp.int32)
                sv = jnp.full((LANES,), s, jnp.int32)
                for k in range(EMB // LANES):
                    w0 = plsc.load_gather(in_v, [sv, te[k], es[k], col - 1])
                    w1 = plsc.load_gather(in_v, [sv, te[k], es[k], col])
                    out_v[s, i, pl.ds(k * LANES, LANES)] = w0
                    out_v[s, i, pl.ds(EMB + k * LANES, LANES)] = w1
                return c

            lax.fori_loop(0, 64, row, jnp.int32(0))
            tv = wid + NW * g
            pltpu.async_copy(out_v.at[s], scr.at[pl.ds(tv * EMB, EMB)],
                             sem_out.at[s])
            return carry

        lax.fori_loop(0, nk, step, jnp.int32(0))
        for j in range(2):
            pltpu.make_async_copy(out_v.at[j], scr.at[pl.ds(0, EMB)],
                                  sem_out.at[j]).wait()

        @pl.when(wid == 4)
        def _(tail=tail, scr=scr):
            pltpu.sync_copy(tail, tail_v)

            def trow(i, c):
                col = jnp.full((LANES,), 2 * i + 1, jnp.int32)
                for k in range(EMB // LANES):
                    w0 = plsc.load_gather(tail_v, [k * LANES + lane, col - 1])
                    w1 = plsc.load_gather(tail_v, [k * LANES + lane, col])
                    out_v[0, i, pl.ds(k * LANES, LANES)] = w0
                    out_v[0, i, pl.ds(EMB + k * LANES, LANES)] = w1
                return c

            lax.fori_loop(0, TAIL // 2, trow, jnp.int32(0))
            pltpu.sync_copy(out_v.at[0, pl.ds(0, TAIL // 2)],
                            scr.at[pl.ds(VT * EMB, TAIL // 2)])


def _gd_body(t_hbm, c_hbm, scr_t, scr_c, out_hbm,
             tidx_v, tidx2_v, cidx_v, cidx2_v, trows_v, crows_v, out_v, sem):
    wid = lax.axis_index("s") * _NC + lax.axis_index("c")
    lane = lax.iota(jnp.int32, LANES)

    for chunk in range(NCHUNK):
        base = wid * BPW + chunk * CB
        pltpu.sync_copy(t_hbm.at[pl.ds(base, CB)], tidx_v)
        pltpu.sync_copy(c_hbm.at[pl.ds(base * CTX, CB * CTX)], cidx_v)
        for s in range(CB // LANES):
            tidx2_v[pl.ds(s * LANES, LANES)] = (
                tidx_v[pl.ds(s * LANES, LANES)] >> 1)
        for j in range(CTX):
            for s in range(CB // LANES):
                cidx2_v[j, pl.ds(s * LANES, LANES)] = (
                    cidx_v[pl.ds(j * CB + s * LANES, LANES)] >> 1)

        copies = [pltpu.async_copy(scr_t.at[tidx2_v], trows_v, sem)]
        for j in range(CTX):
            copies.append(
                pltpu.async_copy(scr_c.at[cidx2_v.at[j]],
                                 crows_v.at[pl.ds(j * CB, CB)], sem))
        for cp in copies:
            cp.wait()

        for g in range(CB // LANES):
            wrow = g * LANES + lane
            wcol0 = (tidx_v[pl.ds(g * LANES, LANES)] & 1) * EMB
            crows = [wrow * CTX + c for c in range(CTX)]
            ccol0 = [(plsc.load_gather(cidx_v, [crows[c]]) & 1) * EMB
                     for c in range(CTX)]

            def body(e, accs, wrow=wrow, wcol0=wcol0, crows=crows, ccol0=ccol0):
                ev = jnp.full((LANES,), e, jnp.int32)
                w = plsc.load_gather(trows_v, [wrow, wcol0 + ev])
                return tuple(
                    accs[c] + w * plsc.load_gather(crows_v,
                                                   [crows[c], ccol0[c] + ev])
                    for c in range(CTX))

            accs = lax.fori_loop(
                0, EMB, body,
                tuple(jnp.zeros((LANES,), jnp.float32) for _ in range(CTX)))
            for c in range(CTX):
                plsc.store_scatter(out_v, [crows[c]], accs[c])

        pltpu.sync_copy(out_v, out_hbm.at[pl.ds(base * CTX, CB * CTX)])


def kernel(target, context, target_table, context_table):
    t = target.reshape(BATCH).astype(jnp.int32)
    c = context.reshape(BATCH * CTX).astype(jnp.int32)
    ttab3 = target_table.T.reshape(8, 8, VOCAB)
    ctab3 = context_table.T.reshape(8, 8, VOCAB)
    ttail = target_table[VT * 128:, :].T
    ctail = context_table[VT * 128:, :].T

    transpose = functools.partial(
        pl.kernel,
        out_type=(jax.ShapeDtypeStruct((PR, 128), jnp.float32),
                  jax.ShapeDtypeStruct((PR, 128), jnp.float32)),
        mesh=_mesh,
        compiler_params=_params,
        scratch_types=[
            pltpu.VMEM((2, 8, 8, 128), jnp.float32),
            pltpu.VMEM((2, EMB, 128), jnp.float32),
            pltpu.VMEM((TAIL, TAIL), jnp.float32),
            pltpu.SemaphoreType.DMA((2,)),
            pltpu.SemaphoreType.DMA((2,)),
        ],
    )(_tr_body)
    scr_t, scr_c = transpose(ttab3, ttail, ctab3, ctail)

    gather_dot = functools.partial(
        pl.kernel,
        out_type=jax.ShapeDtypeStruct((BATCH * CTX,), jnp.float32),
        mesh=_mesh,
        compiler_params=_params,
        scratch_types=[
            pltpu.VMEM((CB,), jnp.int32),
            pltpu.VMEM((CB,), jnp.int32),
            pltpu.VMEM((CB * CTX,), jnp.int32),
            pltpu.VMEM((CTX, CB), jnp.int32),
            pltpu.VMEM((CB, 128), jnp.float32),
            pltpu.VMEM((CB * CTX, 128), jnp.float32),
            pltpu.VMEM((CB * CTX,), jnp.float32),
            pltpu.SemaphoreType.DMA,
        ],
    )(_gd_body)
    dots = gather_dot(t, c, scr_t, scr_c)
    return dots.reshape(BATCH, CTX)


# speedup vs baseline: 1.9726x; 1.9726x over previous
import functools

import jax
import jax.numpy as jnp
from jax import lax
from jax.experimental import pallas as pl
from jax.experimental.pallas import tpu as pltpu
from jax.experimental.pallas import tpu_sc as plsc

VOCAB = 1000000
EMB = 64
BATCH = 16384
CTX = 5
LANES = 16

_info = plsc.get_sparse_core_info()
_NC, _NS = _info.num_cores, _info.num_subcores
NW = _NC * _NS
BPW = BATCH // NW
CB = 128
NCHUNK = BPW // CB

_params = pltpu.CompilerParams(
    needs_layout_passes=False, use_tc_tiling_on_sc=False)
_mesh = plsc.VectorSubcoreMesh(core_axis_name="c", subcore_axis_name="s")


def _gd_body(t_hbm, c_hbm, t_tab, c_tab, out_hbm,
             tidx_v, cidx_v, cidx_m, trows_v, crows_v, out_v, sem):
    wid = lax.axis_index("s") * _NC + lax.axis_index("c")
    lane = lax.iota(jnp.int32, LANES)

    for chunk in range(NCHUNK):
        base = wid * BPW + chunk * CB
        pltpu.sync_copy(t_hbm.at[pl.ds(base, CB)], tidx_v)
        pltpu.sync_copy(c_hbm.at[pl.ds(base * CTX, CB * CTX)], cidx_v)
        for j in range(CTX):
            for s in range(CB // LANES):
                cidx_m[j, pl.ds(s * LANES, LANES)] = (
                    cidx_v[pl.ds(j * CB + s * LANES, LANES)])

        copies = [pltpu.async_copy(t_tab.at[tidx_v], trows_v, sem)]
        for j in range(CTX):
            copies.append(
                pltpu.async_copy(c_tab.at[cidx_m.at[j]],
                                 crows_v.at[pl.ds(j * CB, CB)], sem))
        for cp in copies:
            cp.wait()

        for g in range(CB // LANES):
            wrow = g * LANES + lane
            crows = [wrow * CTX + c for c in range(CTX)]

            def body(e, accs, wrow=wrow, crows=crows):
                ev = jnp.full((LANES,), e, jnp.int32)
                w = plsc.load_gather(trows_v, [wrow, ev])
                return tuple(
                    accs[c] + w * plsc.load_gather(crows_v, [crows[c], ev])
                    for c in range(CTX))

            accs = lax.fori_loop(
                0, EMB, body,
                tuple(jnp.zeros((LANES,), jnp.float32) for _ in range(CTX)))
            for c in range(CTX):
                plsc.store_scatter(out_v, [crows[c]], accs[c])

        pltpu.sync_copy(out_v, out_hbm.at[pl.ds(base * CTX, CB * CTX)])


def kernel(target, context, target_table, context_table):
    t = target.reshape(BATCH).astype(jnp.int32)
    c = context.reshape(BATCH * CTX).astype(jnp.int32)

    gather_dot = functools.partial(
        pl.kernel,
        out_type=jax.ShapeDtypeStruct((BATCH * CTX,), jnp.float32),
        mesh=_mesh,
        compiler_params=_params,
        scratch_types=[
            pltpu.VMEM((CB,), jnp.int32),
            pltpu.VMEM((CB * CTX,), jnp.int32),
            pltpu.VMEM((CTX, CB), jnp.int32),
            pltpu.VMEM((CB, EMB), jnp.float32),
            pltpu.VMEM((CB * CTX, EMB), jnp.float32),
            pltpu.VMEM((CB * CTX,), jnp.float32),
            pltpu.SemaphoreType.DMA,
        ],
    )(_gd_body)
    dots = gather_dot(t, c, target_table, context_table)
    return dots.reshape(BATCH, CTX)
